# Initial kernel scaffold; baseline (speedup 1.0000x reference)
import jax
import jax.numpy as jnp
from jax.experimental import pallas as pl


def kernel(layer_idx, router_weights, num_experts_per_tok, router_logits):
    raise NotImplementedError("write your pallas kernel here")


# speedup vs baseline: 2.4877x; 2.4877x over previous
import functools

import jax
import jax.numpy as jnp
from jax import lax
from jax.experimental import pallas as pl
from jax.experimental.pallas import tpu as pltpu
from jax.experimental.pallas import tpu_sc as plsc

TOKENS = 16384
E = 64
K = 8
L = 16
NC = 2
NS = 16
NW = NC * NS
ROWS = TOKENS // NW

_mesh = plsc.VectorSubcoreMesh(core_axis_name="c", subcore_axis_name="s")


@functools.partial(
    pl.kernel,
    out_type=jax.ShapeDtypeStruct((NW, 3, E), jnp.float32),
    mesh=_mesh,
    compiler_params=pltpu.CompilerParams(needs_layout_passes=False),
    scratch_types=[
        pltpu.VMEM((ROWS, E), jnp.float32),
        pltpu.VMEM((E,), jnp.float32),
        pltpu.VMEM((E,), jnp.float32),
        pltpu.VMEM((3, E), jnp.float32),
    ],
)
def _sc_topk_hist(l_hbm, w_hbm, out_hbm, buf_v, hl_v, hw_v, part_v):
    c = lax.axis_index("c")
    s = lax.axis_index("s")
    wid = s * NC + c
    base = wid * ROWS

    iota = lax.iota(jnp.int32, L)
    zeros = jnp.zeros((L,), jnp.float32)
    ones = jnp.ones((L,), jnp.float32)
    top8_mask = iota < K
    idx_consts = [iota + L * j for j in range(E // L)]
    for j in range(E // L):
        hl_v[pl.ds(L * j, L)] = zeros
        hw_v[pl.ds(L * j, L)] = zeros

    def merge(ka, va, kb, vb):
        rk = lax.rev(kb, (0,))
        rv = lax.rev(vb, (0,))
        take_a = ka >= rk
        mk = jnp.maximum(ka, rk)
        mv = jnp.where(take_a, va, rv)
        return plsc.sort_key_val(mk, mv, descending=True)

    def top8_into(src_v, hist_ref, r):
        ks, vs = [], []
        for j in range(E // L):
            k_s, v_s = plsc.sort_key_val(
                src_v[r, pl.ds(L * j, L)], idx_consts[j], descending=True
            )
            ks.append(k_s)
            vs.append(v_s)
        k01, v01 = merge(ks[0], vs[0], ks[1], vs[1])
        k23, v23 = merge(ks[2], vs[2], ks[3], vs[3])
        _, vf = merge(k01, v01, k23, v23)
        plsc.addupdate_scatter(hist_ref, [vf], ones, mask=top8_mask)

    pltpu.sync_copy(l_hbm.at[pl.ds(base, ROWS)], buf_v)

    def body_l(r, carry):
        top8_into(buf_v, hl_v, r)
        return carry

    lax.fori_loop(0, ROWS, body_l, 0)

    pltpu.sync_copy(w_hbm.at[pl.ds(base, ROWS)], buf_v)

    def body_w(r, carry):
        top8_into(buf_v, hw_v, r)
        return tuple(
            acc + buf_v[r, pl.ds(L * j, L)] for j, acc in enumerate(carry)
        )

    sums = lax.fori_loop(0, ROWS, body_w, (zeros,) * (E // L))

    for j in range(E // L):
        part_v[0, pl.ds(L * j, L)] = hl_v[pl.ds(L * j, L)]
        part_v[1, pl.ds(L * j, L)] = hw_v[pl.ds(L * j, L)]
        part_v[2, pl.ds(L * j, L)] = sums[j]
    pltpu.sync_copy(part_v, out_hbm.at[wid])


def _reduce_body(x_ref, o_ref):
    o_ref[...] = jnp.sum(x_ref[...], axis=0, keepdims=True)


def kernel(layer_idx, router_weights, num_experts_per_tok, router_logits):
    partials = _sc_topk_hist(
        router_logits.astype(jnp.float32), router_weights.astype(jnp.float32)
    )
    flat = partials.reshape(NW, 3 * E)
    out = pl.pallas_call(
        _reduce_body,
        out_shape=jax.ShapeDtypeStruct((1, 3 * E), jnp.float32),
    )(flat)
    return out.reshape(3, E)


# speedup vs baseline: 4.2327x; 1.7014x over previous
import functools

import jax
import jax.numpy as jnp
from jax import lax
from jax.experimental import pallas as pl
from jax.experimental.pallas import tpu as pltpu
from jax.experimental.pallas import tpu_sc as plsc

TOKENS = 16384
E = 64
K = 8
L = 16
NC = 2
NS = 16
NW = NC * NS
ROWS = TOKENS // NW

_mesh = plsc.VectorSubcoreMesh(core_axis_name="c", subcore_axis_name="s")


@functools.partial(
    pl.kernel,
    out_type=jax.ShapeDtypeStruct((NW, 3, E), jnp.float32),
    mesh=_mesh,
    compiler_params=pltpu.CompilerParams(needs_layout_passes=False),
    scratch_types=[
        pltpu.VMEM((ROWS, E), jnp.float32),
        pltpu.VMEM((E,), jnp.float32),
        pltpu.VMEM((E,), jnp.float32),
        pltpu.VMEM((3, E), jnp.float32),
    ],
)
def _sc_topk_hist(l_hbm, w_hbm, out_hbm, buf_v, hl_v, hw_v, part_v):
    c = lax.axis_index("c")
    s = lax.axis_index("s")
    wid = s * NC + c
    base = wid * ROWS

    iota = lax.iota(jnp.int32, L)
    zeros = jnp.zeros((L,), jnp.float32)
    ones = jnp.ones((L,), jnp.float32)
    top8_mask = iota < K
    idx_consts = [iota + L * j for j in range(E // L)]
    for j in range(E // L):
        hl_v[pl.ds(L * j, L)] = zeros
        hw_v[pl.ds(L * j, L)] = zeros

    def merge(ka, va, kb, vb):
        rk = lax.rev(kb, (0,))
        rv = lax.rev(vb, (0,))
        take_a = ka >= rk
        mk = jnp.maximum(ka, rk)
        mv = jnp.where(take_a, va, rv)
        return plsc.sort_key_val(mk, mv, descending=True)

    def top8_into(src_v, hist_ref, r):
        ks, vs = [], []
        for j in range(E // L):
            k_s, v_s = plsc.sort_key_val(
                src_v[r, pl.ds(L * j, L)], idx_consts[j], descending=True
            )
            ks.append(k_s)
            vs.append(v_s)
        k01, v01 = merge(ks[0], vs[0], ks[1], vs[1])
        k23, v23 = merge(ks[2], vs[2], ks[3], vs[3])
        _, vf = merge(k01, v01, k23, v23)
        plsc.addupdate_scatter(hist_ref, [vf], ones, mask=top8_mask)

    pltpu.sync_copy(l_hbm.at[pl.ds(base, ROWS)], buf_v)

    @plsc.parallel_loop(0, ROWS, unroll=4)
    def _(r):
        top8_into(buf_v, hl_v, r)

    pltpu.sync_copy(w_hbm.at[pl.ds(base, ROWS)], buf_v)

    @plsc.parallel_loop(0, ROWS, unroll=4, carry=(zeros,) * (E // L))
    def sums(r, carry):
        top8_into(buf_v, hw_v, r)
        return tuple(
            acc + buf_v[r, pl.ds(L * j, L)] for j, acc in enumerate(carry)
        )

    for j in range(E // L):
        part_v[0, pl.ds(L * j, L)] = hl_v[pl.ds(L * j, L)]
        part_v[1, pl.ds(L * j, L)] = hw_v[pl.ds(L * j, L)]
        part_v[2, pl.ds(L * j, L)] = sums[j]
    pltpu.sync_copy(part_v, out_hbm.at[wid])


def _reduce_body(x_ref, o_ref):
    o_ref[...] = jnp.sum(x_ref[...], axis=0, keepdims=True)


def kernel(layer_idx, router_weights, num_experts_per_tok, router_logits):
    partials = _sc_topk_hist(
        router_logits.astype(jnp.float32), router_weights.astype(jnp.float32)
    )
    flat = partials.reshape(NW, 3 * E)
    out = pl.pallas_call(
        _reduce_body,
        out_shape=jax.ShapeDtypeStruct((1, 3 * E), jnp.float32),
    )(flat)
    return out.reshape(3, E)


# speedup vs baseline: 4.2409x; 1.0019x over previous
import functools

import jax
import jax.numpy as jnp
from jax import lax
from jax.experimental import pallas as pl
from jax.experimental.pallas import tpu as pltpu
from jax.experimental.pallas import tpu_sc as plsc

TOKENS = 16384
E = 64
K = 8
L = 16
NC = 2
NS = 16
NW = NC * NS
ROWS = TOKENS // NW

_mesh = plsc.VectorSubcoreMesh(core_axis_name="c", subcore_axis_name="s")


@functools.partial(
    pl.kernel,
    out_type=jax.ShapeDtypeStruct((NW, 3, E), jnp.float32),
    mesh=_mesh,
    compiler_params=pltpu.CompilerParams(needs_layout_passes=False, use_tc_tiling_on_sc=True),
    scratch_types=[
        pltpu.VMEM((ROWS, E), jnp.float32),
        pltpu.VMEM((E,), jnp.float32),
        pltpu.VMEM((E,), jnp.float32),
        pltpu.VMEM((3, E), jnp.float32),
    ],
)
def _sc_topk_hist(l_hbm, w_hbm, out_hbm, buf_v, hl_v, hw_v, part_v):
    c = lax.axis_index("c")
    s = lax.axis_index("s")
    wid = s * NC + c
    base = wid * ROWS

    iota = lax.iota(jnp.int32, L)
    zeros = jnp.zeros((L,), jnp.float32)
    ones = jnp.ones((L,), jnp.float32)
    top8_mask = iota < K
    idx_consts = [iota + L * j for j in range(E // L)]
    for j in range(E // L):
        hl_v[pl.ds(L * j, L)] = zeros
        hw_v[pl.ds(L * j, L)] = zeros

    def merge(ka, va, kb, vb):
        rk = lax.rev(kb, (0,))
        rv = lax.rev(vb, (0,))
        take_a = ka >= rk
        mk = jnp.maximum(ka, rk)
        mv = jnp.where(take_a, va, rv)
        return plsc.sort_key_val(mk, mv, descending=True)

    def top8_into(src_v, hist_ref, r):
        ks, vs = [], []
        for j in range(E // L):
            k_s, v_s = plsc.sort_key_val(
                src_v[r, pl.ds(L * j, L)], idx_consts[j], descending=True
            )
            ks.append(k_s)
            vs.append(v_s)
        k01, v01 = merge(ks[0], vs[0], ks[1], vs[1])
        k23, v23 = merge(ks[2], vs[2], ks[3], vs[3])
        _, vf = merge(k01, v01, k23, v23)
        plsc.addupdate_scatter(hist_ref, [vf], ones, mask=top8_mask)

    pltpu.sync_copy(l_hbm.at[pl.ds(base, ROWS)], buf_v)

    @plsc.parallel_loop(0, ROWS, unroll=4)
    def _(r):
        top8_into(buf_v, hl_v, r)

    pltpu.sync_copy(w_hbm.at[pl.ds(base, ROWS)], buf_v)

    @plsc.parallel_loop(0, ROWS, unroll=4, carry=(zeros,) * (E // L))
    def sums(r, carry):
        top8_into(buf_v, hw_v, r)
        return tuple(
            acc + buf_v[r, pl.ds(L * j, L)] for j, acc in enumerate(carry)
        )

    for j in range(E // L):
        part_v[0, pl.ds(L * j, L)] = hl_v[pl.ds(L * j, L)]
        part_v[1, pl.ds(L * j, L)] = hw_v[pl.ds(L * j, L)]
        part_v[2, pl.ds(L * j, L)] = sums[j]
    pltpu.sync_copy(part_v, out_hbm.at[wid])


def _reduce_body(x_ref, o_ref):
    o_ref[...] = jnp.sum(x_ref[...], axis=0, keepdims=True)


def kernel(layer_idx, router_weights, num_experts_per_tok, router_logits):
    partials = _sc_topk_hist(
        router_logits.astype(jnp.float32), router_weights.astype(jnp.float32)
    )
    flat = partials.reshape(NW, 3 * E)
    out = pl.pallas_call(
        _reduce_body,
        out_shape=jax.ShapeDtypeStruct((1, 3 * E), jnp.float32),
    )(flat)
    return out.reshape(3, E)
